# Initial kernel scaffold; baseline (speedup 1.0000x reference)
import functools

import jax
import jax.numpy as jnp
from jax import lax
from jax.experimental import pallas as pl
from jax.experimental.pallas import tpu as pltpu
from jax.experimental.pallas import tpu_sc as plsc

N = 10000
D = 128
F = 64
G = 64
OUT = 64
E = 320000

NT = 10240
ROWS_PER_SUB = NT // 16
CHUNK = 128
NW = 32
EP = E + N
T_CH = -(-EP // (NW * CHUNK))
EPAD = NW * T_CH * CHUNK


def _prep1_body(x_ref, w_ref, as_ref, ad_ref, h_ref, asv_ref, adv_ref):
    x = x_ref[...]
    h = jnp.dot(x, w_ref[...], preferred_element_type=jnp.float32)
    h_ref[...] = h
    asv_ref[...] = jnp.dot(h, as_ref[...], preferred_element_type=jnp.float32)
    adv_ref[...] = jnp.dot(h, ad_ref[...], preferred_element_type=jnp.float32)


def _prep_next_body(np_ref, dp_ref, b_ref, w_ref, as_ref, ad_ref,
                    x_ref, h_ref, asv_ref, adv_ref):
    num = np_ref[0] + np_ref[1]
    den = dp_ref[0] + dp_ref[1]
    x = jnp.maximum(num / (den + 1e-16) + b_ref[...], 0.0)
    x_ref[...] = x
    h = jnp.dot(x, w_ref[...], preferred_element_type=jnp.float32)
    h_ref[...] = h
    asv_ref[...] = jnp.dot(h, as_ref[...], preferred_element_type=jnp.float32)
    adv_ref[...] = jnp.dot(h, ad_ref[...], preferred_element_type=jnp.float32)


def _final_body(np_ref, dp_ref, b_ref, x1_ref, x2_ref, batch_ref,
                wl_ref, bl_ref, out_ref):
    num = np_ref[0] + np_ref[1]
    den = dp_ref[0] + dp_ref[1]
    x3 = jnp.maximum(num / (den + 1e-16) + b_ref[...], 0.0)
    y = (x1_ref[...] + x2_ref[...] + x3) * (1.0 / 3.0)
    onehot = (batch_ref[...] == lax.broadcasted_iota(jnp.int32, (NT, G), 1)
              ).astype(jnp.float32)
    cdims = (((0,), (0,)), ((), ()))
    sums = lax.dot_general(onehot, y, cdims, preferred_element_type=jnp.float32)
    counts = lax.dot_general(onehot, jnp.ones((NT, 1), jnp.float32), cdims,
                             preferred_element_type=jnp.float32)
    pooled = sums / jnp.maximum(counts, 1.0)
    logits = jnp.dot(pooled, wl_ref[...], preferred_element_type=jnp.float32)
    logits = logits + bl_ref[...]
    m = jnp.max(logits, axis=1, keepdims=True)
    z = jnp.exp(logits - m)
    out_ref[...] = z / jnp.sum(z, axis=1, keepdims=True)


def _prep1(x, W, a_s, a_d):
    return pl.pallas_call(
        _prep1_body,
        out_shape=(jax.ShapeDtypeStruct((NT, F), jnp.float32),
                   jax.ShapeDtypeStruct((NT, 1), jnp.float32),
                   jax.ShapeDtypeStruct((NT, 1), jnp.float32)),
    )(x, W, a_s, a_d)


def _prep_next(num_p, den_p, b, W, a_s, a_d):
    return pl.pallas_call(
        _prep_next_body,
        out_shape=(jax.ShapeDtypeStruct((NT, F), jnp.float32),
                   jax.ShapeDtypeStruct((NT, F), jnp.float32),
                   jax.ShapeDtypeStruct((NT, 1), jnp.float32),
                   jax.ShapeDtypeStruct((NT, 1), jnp.float32)),
    )(num_p, den_p, b, W, a_s, a_d)


def _final(num_p, den_p, b, x1, x2, batch_col, Wl, bl):
    return pl.pallas_call(
        _final_body,
        out_shape=jax.ShapeDtypeStruct((G, OUT), jnp.float32),
    )(num_p, den_p, b, x1, x2, batch_col, Wl, bl)


_MESH = plsc.VectorSubcoreMesh(core_axis_name="c", subcore_axis_name="s")


def _edge_body(src_hbm, dst_hbm, h_hbm, asv_hbm, adv_hbm, z2_hbm, z1_hbm,
               num_out, den_out,
               asv_v, adv_v, srcv, dstv, wv, rows, num_sp, den_sp, sem):
    cid = lax.axis_index("c")
    sid = lax.axis_index("s")
    wid = sid * 2 + cid
    base_n = sid * ROWS_PER_SUB

    pltpu.sync_copy(z2_hbm, num_sp.at[pl.ds(base_n, ROWS_PER_SUB)])
    pltpu.sync_copy(z1_hbm, den_sp.at[pl.ds(base_n, ROWS_PER_SUB)])
    pltpu.sync_copy(asv_hbm, asv_v)
    pltpu.sync_copy(adv_hbm, adv_v)
    plsc.subcore_barrier()

    def chunk_body(t, carry):
        base = (wid * T_CH + t) * CHUNK
        pltpu.sync_copy(src_hbm.at[pl.ds(base, CHUNK)], srcv)
        pltpu.sync_copy(dst_hbm.at[pl.ds(base, CHUNK)], dstv)
        cp = pltpu.async_copy(h_hbm.at[srcv], rows, sem)
        for j in range(CHUNK // 16):
            si = srcv[pl.ds(j * 16, 16)]
            di = dstv[pl.ds(j * 16, 16)]
            e = plsc.load_gather(asv_v, [si]) + plsc.load_gather(adv_v, [di])
            e = jnp.maximum(e, 0.2 * e)
            wv[pl.ds(j * 16, 16)] = jnp.exp(e)
        cp.wait()

        def scale_body(i, c2):
            wb = jnp.full((16,), wv[i], jnp.float32)
            for q in range(F // 16):
                rows[i, pl.ds(q * 16, 16)] = rows[i, pl.ds(q * 16, 16)] * wb
            return c2

        lax.fori_loop(0, CHUNK, scale_body, 0)
        pltpu.sync_copy(rows, num_sp.at[dstv], add=True)
        pltpu.sync_copy(wv, den_sp.at[dstv], add=True)
        return carry

    lax.fori_loop(0, T_CH, chunk_body, 0)
    plsc.subcore_barrier()
    pltpu.sync_copy(num_sp.at[pl.ds(base_n, ROWS_PER_SUB)],
                    num_out.at[cid, pl.ds(base_n, ROWS_PER_SUB)])
    pltpu.sync_copy(den_sp.at[pl.ds(base_n, ROWS_PER_SUB)],
                    den_out.at[cid, pl.ds(base_n, ROWS_PER_SUB)])


_edge_agg = functools.partial(
    pl.kernel,
    mesh=_MESH,
    out_type=(jax.ShapeDtypeStruct((2, NT, F), jnp.float32),
              jax.ShapeDtypeStruct((2, NT), jnp.float32)),
    scratch_types=[
        pltpu.VMEM((NT,), jnp.float32),
        pltpu.VMEM((NT,), jnp.float32),
        pltpu.VMEM((CHUNK,), jnp.int32),
        pltpu.VMEM((CHUNK,), jnp.int32),
        pltpu.VMEM((CHUNK,), jnp.float32),
        pltpu.VMEM((CHUNK, F), jnp.float32),
        pltpu.VMEM_SHARED((NT, F), jnp.float32),
        pltpu.VMEM_SHARED((NT,), jnp.float32),
        pltpu.SemaphoreType.DMA,
    ],
)(_edge_body)


def kernel(X, L, batch, W1, a_s1, a_d1, b1, W2, a_s2, a_d2, b2,
           W3, a_s3, a_d3, b3, Wl, bl):
    ei = L[0]
    loop = jnp.arange(N, dtype=ei.dtype)
    src = jnp.concatenate([ei[0], loop])
    dst = jnp.concatenate([ei[1], loop])
    src = jnp.pad(src, (0, EPAD - EP), constant_values=N)
    dst = jnp.pad(dst, (0, EPAD - EP), constant_values=N)

    x0 = jnp.pad(X[0], ((0, NT - N), (0, 0)))
    batch_col = jnp.pad(batch[0], (0, NT - N), constant_values=G).reshape(NT, 1)
    z2 = jnp.zeros((ROWS_PER_SUB, F), jnp.float32)
    z1 = jnp.zeros((ROWS_PER_SUB,), jnp.float32)

    def layer_edges(h, asv, adv):
        return _edge_agg(src, dst, h, asv.reshape(NT), adv.reshape(NT), z2, z1)

    h1, asv1, adv1 = _prep1(x0, W1, a_s1.reshape(F, 1), a_d1.reshape(F, 1))
    np1, dn1 = layer_edges(h1, asv1, adv1)
    x1, h2, asv2, adv2 = _prep_next(np1, dn1.reshape(2, NT, 1),
                                    b1.reshape(1, F), W2,
                                    a_s2.reshape(F, 1), a_d2.reshape(F, 1))
    np2, dn2 = layer_edges(h2, asv2, adv2)
    x2, h3, asv3, adv3 = _prep_next(np2, dn2.reshape(2, NT, 1),
                                    b2.reshape(1, F), W3,
                                    a_s3.reshape(F, 1), a_d3.reshape(F, 1))
    np3, dn3 = layer_edges(h3, asv3, adv3)
    return _final(np3, dn3.reshape(2, NT, 1), b3.reshape(1, F),
                  x1, x2, batch_col, Wl, bl)


# speedup vs baseline: 28.9286x; 28.9286x over previous
import functools

import jax
import jax.numpy as jnp
from jax import lax
from jax.experimental import pallas as pl
from jax.experimental.pallas import tpu as pltpu
from jax.experimental.pallas import tpu_sc as plsc

N = 10000
D = 128
F = 64
G = 64
OUT = 64
E = 320000

NT = 10240
ROWS_PER_SUB = NT // 16
CHUNK = 128
NW = 32
EP = E + N
T_CH = -(-EP // (NW * CHUNK))
EPAD = NW * T_CH * CHUNK


def _prep1_body(x_ref, w_ref, as_ref, ad_ref, h_ref, asv_ref, adv_ref):
    x = x_ref[...]
    h = jnp.dot(x, w_ref[...], preferred_element_type=jnp.float32)
    h_ref[...] = h
    asv_ref[...] = jnp.dot(h, as_ref[...], preferred_element_type=jnp.float32)
    adv_ref[...] = jnp.dot(h, ad_ref[...], preferred_element_type=jnp.float32)


def _prep_next_body(np_ref, dp_ref, b_ref, w_ref, as_ref, ad_ref,
                    x_ref, h_ref, asv_ref, adv_ref):
    num = np_ref[0] + np_ref[1]
    den = dp_ref[0] + dp_ref[1]
    x = jnp.maximum(num / (den + 1e-16) + b_ref[...], 0.0)
    x_ref[...] = x
    h = jnp.dot(x, w_ref[...], preferred_element_type=jnp.float32)
    h_ref[...] = h
    asv_ref[...] = jnp.dot(h, as_ref[...], preferred_element_type=jnp.float32)
    adv_ref[...] = jnp.dot(h, ad_ref[...], preferred_element_type=jnp.float32)


def _final_body(np_ref, dp_ref, b_ref, x1_ref, x2_ref, batch_ref,
                wl_ref, bl_ref, out_ref):
    num = np_ref[0] + np_ref[1]
    den = dp_ref[0] + dp_ref[1]
    x3 = jnp.maximum(num / (den + 1e-16) + b_ref[...], 0.0)
    y = (x1_ref[...] + x2_ref[...] + x3) * (1.0 / 3.0)
    onehot = (batch_ref[...] == lax.broadcasted_iota(jnp.int32, (NT, G), 1)
              ).astype(jnp.float32)
    cdims = (((0,), (0,)), ((), ()))
    sums = lax.dot_general(onehot, y, cdims, preferred_element_type=jnp.float32)
    counts = lax.dot_general(onehot, jnp.ones((NT, 1), jnp.float32), cdims,
                             preferred_element_type=jnp.float32)
    pooled = sums / jnp.maximum(counts, 1.0)
    logits = jnp.dot(pooled, wl_ref[...], preferred_element_type=jnp.float32)
    logits = logits + bl_ref[...]
    m = jnp.max(logits, axis=1, keepdims=True)
    z = jnp.exp(logits - m)
    out_ref[...] = z / jnp.sum(z, axis=1, keepdims=True)


def _prep1(x, W, a_s, a_d):
    return pl.pallas_call(
        _prep1_body,
        out_shape=(jax.ShapeDtypeStruct((NT, F), jnp.float32),
                   jax.ShapeDtypeStruct((NT, 1), jnp.float32),
                   jax.ShapeDtypeStruct((NT, 1), jnp.float32)),
    )(x, W, a_s, a_d)


def _prep_next(num_p, den_p, b, W, a_s, a_d):
    return pl.pallas_call(
        _prep_next_body,
        out_shape=(jax.ShapeDtypeStruct((NT, F), jnp.float32),
                   jax.ShapeDtypeStruct((NT, F), jnp.float32),
                   jax.ShapeDtypeStruct((NT, 1), jnp.float32),
                   jax.ShapeDtypeStruct((NT, 1), jnp.float32)),
    )(num_p, den_p, b, W, a_s, a_d)


def _final(num_p, den_p, b, x1, x2, batch_col, Wl, bl):
    return pl.pallas_call(
        _final_body,
        out_shape=jax.ShapeDtypeStruct((G, OUT), jnp.float32),
    )(num_p, den_p, b, x1, x2, batch_col, Wl, bl)


_MESH = plsc.VectorSubcoreMesh(core_axis_name="c", subcore_axis_name="s")


def _edge_body(src_hbm, dst_hbm, h_hbm, asv_hbm, adv_hbm, z2_hbm, z1_hbm,
               num_out, den_out,
               asv_v, adv_v, srcv, dstv, wv, rows, num_sp, den_sp, sem):
    cid = lax.axis_index("c")
    sid = lax.axis_index("s")
    wid = sid * 2 + cid
    base_n = sid * ROWS_PER_SUB

    pltpu.sync_copy(z2_hbm, num_sp.at[pl.ds(base_n, ROWS_PER_SUB)])
    pltpu.sync_copy(z1_hbm, den_sp.at[pl.ds(base_n, ROWS_PER_SUB)])
    pltpu.sync_copy(asv_hbm, asv_v)
    pltpu.sync_copy(adv_hbm, adv_v)
    plsc.subcore_barrier()

    def chunk_body(t, carry):
        base = (wid * T_CH + t) * CHUNK
        pltpu.sync_copy(src_hbm.at[pl.ds(base, CHUNK)], srcv)
        pltpu.sync_copy(dst_hbm.at[pl.ds(base, CHUNK)], dstv)
        cp = pltpu.async_copy(h_hbm.at[srcv], rows, sem)
        for j in range(CHUNK // 16):
            si = srcv[pl.ds(j * 16, 16)]
            di = dstv[pl.ds(j * 16, 16)]
            e = plsc.load_gather(asv_v, [si]) + plsc.load_gather(adv_v, [di])
            e = jnp.maximum(e, 0.2 * e)
            wv[pl.ds(j * 16, 16)] = jnp.exp(e)
        cp.wait()

        def scale_body(i, c2):
            wb = plsc.load_gather(wv, [jnp.full((16,), i, jnp.int32)])
            for q in range(F // 16):
                rows[i, pl.ds(q * 16, 16)] = rows[i, pl.ds(q * 16, 16)] * wb
            return c2

        lax.fori_loop(0, CHUNK, scale_body, 0)
        pltpu.sync_copy(rows, num_sp.at[dstv], add=True)
        pltpu.sync_copy(wv, den_sp.at[dstv], add=True)
        return carry

    lax.fori_loop(0, T_CH, chunk_body, 0)
    plsc.subcore_barrier()
    pltpu.sync_copy(num_sp.at[pl.ds(base_n, ROWS_PER_SUB)],
                    num_out.at[cid, pl.ds(base_n, ROWS_PER_SUB)])
    pltpu.sync_copy(den_sp.at[pl.ds(base_n, ROWS_PER_SUB)],
                    den_out.at[cid, pl.ds(base_n, ROWS_PER_SUB)])


_edge_agg = functools.partial(
    pl.kernel,
    mesh=_MESH,
    compiler_params=pltpu.CompilerParams(needs_layout_passes=False,
                                         use_tc_tiling_on_sc=False),
    out_type=(jax.ShapeDtypeStruct((2, NT, F), jnp.float32),
              jax.ShapeDtypeStruct((2, NT), jnp.float32)),
    scratch_types=[
        pltpu.VMEM((NT,), jnp.float32),
        pltpu.VMEM((NT,), jnp.float32),
        pltpu.VMEM((CHUNK,), jnp.int32),
        pltpu.VMEM((CHUNK,), jnp.int32),
        pltpu.VMEM((CHUNK,), jnp.float32),
        pltpu.VMEM((CHUNK, F), jnp.float32),
        pltpu.VMEM_SHARED((NT, F), jnp.float32),
        pltpu.VMEM_SHARED((NT,), jnp.float32),
        pltpu.SemaphoreType.DMA,
    ],
)(_edge_body)


def kernel(X, L, batch, W1, a_s1, a_d1, b1, W2, a_s2, a_d2, b2,
           W3, a_s3, a_d3, b3, Wl, bl):
    ei = L[0]
    loop = jnp.arange(N, dtype=ei.dtype)
    src = jnp.concatenate([ei[0], loop])
    dst = jnp.concatenate([ei[1], loop])
    src = jnp.pad(src, (0, EPAD - EP), constant_values=N)
    dst = jnp.pad(dst, (0, EPAD - EP), constant_values=N)

    x0 = jnp.pad(X[0], ((0, NT - N), (0, 0)))
    batch_col = jnp.pad(batch[0], (0, NT - N), constant_values=G).reshape(NT, 1)
    z2 = jnp.zeros((ROWS_PER_SUB, F), jnp.float32)
    z1 = jnp.zeros((ROWS_PER_SUB,), jnp.float32)

    def layer_edges(h, asv, adv):
        return _edge_agg(src, dst, h, asv.reshape(NT), adv.reshape(NT), z2, z1)

    h1, asv1, adv1 = _prep1(x0, W1, a_s1.reshape(F, 1), a_d1.reshape(F, 1))
    np1, dn1 = layer_edges(h1, asv1, adv1)
    x1, h2, asv2, adv2 = _prep_next(np1, dn1.reshape(2, NT, 1),
                                    b1.reshape(1, F), W2,
                                    a_s2.reshape(F, 1), a_d2.reshape(F, 1))
    np2, dn2 = layer_edges(h2, asv2, adv2)
    x2, h3, asv3, adv3 = _prep_next(np2, dn2.reshape(2, NT, 1),
                                    b2.reshape(1, F), W3,
                                    a_s3.reshape(F, 1), a_d3.reshape(F, 1))
    np3, dn3 = layer_edges(h3, asv3, adv3)
    return _final(np3, dn3.reshape(2, NT, 1), b3.reshape(1, F),
                  x1, x2, batch_col, Wl, bl)
